# baseline (speedup 1.0000x reference)
import functools

import jax
import jax.numpy as jnp
from jax import lax
from jax.experimental import pallas as pl
from jax.experimental.pallas import tpu as pltpu
from jax.experimental.pallas import tpu_sc as plsc

T = 512
ROW = 3 * 224 * 224
NKEY = 171
NW = 32
PER_W = -(-NKEY // NW)


def kernel(video):
    v2 = video.reshape(T, ROW)
    mesh = plsc.VectorSubcoreMesh(core_axis_name="c", subcore_axis_name="s")

    @functools.partial(
        pl.kernel,
        mesh=mesh,
        out_type=jax.ShapeDtypeStruct((NKEY, ROW), jnp.float32),
        scratch_types=[pltpu.SemaphoreType.DMA],
    )
    def k(v_hbm, o_hbm, sem):
        wid = lax.axis_index("s") * 2 + lax.axis_index("c")
        for j in range(PER_W):
            f = j * NW + wid

            @pl.when(f < NKEY)
            def _():
                src = jnp.maximum(3 * f - 1, 0)
                pltpu.make_async_copy(v_hbm.at[src], o_hbm.at[f], sem).start()

        for j in range(PER_W):
            f = j * NW + wid

            @pl.when(f < NKEY)
            def _():
                pltpu.make_async_copy(v_hbm.at[0], o_hbm.at[0], sem).wait()

    out = k(v2)
    return out.reshape(NKEY, 3, 224, 224)


# speedup vs baseline: 6.7899x; 6.7899x over previous
import functools

import jax
import jax.numpy as jnp
from jax import lax
from jax.experimental import pallas as pl
from jax.experimental.pallas import tpu as pltpu
from jax.experimental.pallas import tpu_sc as plsc

T = 512
ROW = 3 * 224 * 224
NKEY = 171
NW = 32
CHUNKS = 4
C = ROW // CHUNKS
TOTAL = NKEY * CHUNKS
NT = -(-TOTAL // NW)
NBUF = 3


def kernel(video):
    v2 = video.reshape(T, ROW)
    mesh = plsc.VectorSubcoreMesh(core_axis_name="c", subcore_axis_name="s")

    @functools.partial(
        pl.kernel,
        mesh=mesh,
        out_type=jax.ShapeDtypeStruct((NKEY, ROW), jnp.float32),
        scratch_types=(
            [pltpu.VMEM((C,), jnp.float32)] * NBUF
            + [pltpu.SemaphoreType.DMA] * (2 * NBUF)
        ),
    )
    def k(v_hbm, o_hbm, *scratch):
        bufs = scratch[:NBUF]
        gsems = scratch[NBUF:2 * NBUF]
        ssems = scratch[2 * NBUF:]
        wid = lax.axis_index("s") * 2 + lax.axis_index("c")

        def slices(t):
            u = t * NW + wid
            f = u // CHUNKS
            co = (u % CHUNKS) * C
            src = jnp.maximum(3 * f - 1, 0)
            return v_hbm.at[src, pl.ds(co, C)], o_hbm.at[f, pl.ds(co, C)]

        def guarded(t, op):
            if (NT - 1) * NW + NW - 1 < TOTAL or t < NT - 1:
                op()
            else:
                pl.when(t * NW + wid < TOTAL)(op)

        def g_start(t):
            s, _ = slices(t)
            guarded(t, lambda: pltpu.make_async_copy(
                s, bufs[t % NBUF], gsems[t % NBUF]).start())

        def g_wait(t):
            s, _ = slices(t)
            guarded(t, lambda: pltpu.make_async_copy(
                s, bufs[t % NBUF], gsems[t % NBUF]).wait())

        def s_start(t):
            _, d = slices(t)
            guarded(t, lambda: pltpu.make_async_copy(
                bufs[t % NBUF], d, ssems[t % NBUF]).start())

        def s_wait(t):
            _, d = slices(t)
            guarded(t, lambda: pltpu.make_async_copy(
                bufs[t % NBUF], d, ssems[t % NBUF]).wait())

        for t in range(NBUF - 1):
            g_start(t)
        for t in range(NT):
            if t + NBUF - 1 < NT:
                if t - 1 >= 0:
                    s_wait(t - 1)
                g_start(t + NBUF - 1)
            g_wait(t)
            s_start(t)
        for t in range(max(0, NT - NBUF), NT):
            s_wait(t)

    out = k(v2)
    return out.reshape(NKEY, 3, 224, 224)


# speedup vs baseline: 8.7285x; 1.2855x over previous
import functools

import jax
import jax.numpy as jnp
from jax import lax
from jax.experimental import pallas as pl
from jax.experimental.pallas import tpu as pltpu
from jax.experimental.pallas import tpu_sc as plsc

T = 512
CH = 3
H = 224
W = 224
NKEY = 171
NW = 32
TOTAL = NKEY * CH
NT = -(-TOTAL // NW)
NBUF = 2


def kernel(video):
    mesh = plsc.VectorSubcoreMesh(core_axis_name="c", subcore_axis_name="s")

    @functools.partial(
        pl.kernel,
        mesh=mesh,
        out_type=jax.ShapeDtypeStruct((NKEY, CH, H, W), jnp.float32),
        scratch_types=(
            [pltpu.VMEM((H, W), jnp.float32)] * NBUF
            + [pltpu.SemaphoreType.DMA] * (2 * NBUF)
        ),
        compiler_params=pltpu.CompilerParams(use_tc_tiling_on_sc=True),
    )
    def k(v_hbm, o_hbm, *scratch):
        bufs = scratch[:NBUF]
        gsems = scratch[NBUF:2 * NBUF]
        ssems = scratch[2 * NBUF:]
        wid = lax.axis_index("s") * 2 + lax.axis_index("c")

        def slices(t):
            u = t * NW + wid
            f = u // CH
            c = u % CH
            src = jnp.maximum(3 * f - 1, 0)
            return v_hbm.at[src, c], o_hbm.at[f, c]

        def guarded(t, op):
            if (NT - 1) * NW + NW - 1 < TOTAL or t < NT - 1:
                op()
            else:
                pl.when(t * NW + wid < TOTAL)(op)

        def g_start(t):
            s, _ = slices(t)
            guarded(t, lambda: pltpu.make_async_copy(
                s, bufs[t % NBUF], gsems[t % NBUF]).start())

        def g_wait(t):
            s, _ = slices(t)
            guarded(t, lambda: pltpu.make_async_copy(
                s, bufs[t % NBUF], gsems[t % NBUF]).wait())

        def s_start(t):
            _, d = slices(t)
            guarded(t, lambda: pltpu.make_async_copy(
                bufs[t % NBUF], d, ssems[t % NBUF]).start())

        def s_wait(t):
            _, d = slices(t)
            guarded(t, lambda: pltpu.make_async_copy(
                bufs[t % NBUF], d, ssems[t % NBUF]).wait())

        for t in range(NBUF - 1):
            g_start(t)
        for t in range(NT):
            if t + NBUF - 1 < NT:
                if t - 1 >= 0:
                    s_wait(t - 1)
                g_start(t + NBUF - 1)
            g_wait(t)
            s_start(t)
        for t in range(max(0, NT - NBUF), NT):
            s_wait(t)

    return k(video)
